# Initial kernel scaffold; baseline (speedup 1.0000x reference)
import jax
import jax.numpy as jnp
from jax.experimental import pallas as pl


def kernel(tokens, embed, Wq, bq, Wk, bk, Wv, bv, Wo, bo, g1, beta1, W1, c1, W2, c2, g2, beta2):
    raise NotImplementedError("write your pallas kernel here")


# speedup vs baseline: 7.4766x; 7.4766x over previous
import functools

import numpy as np
import jax
import jax.numpy as jnp
from jax.experimental import pallas as pl
from jax.experimental.pallas import tpu as pltpu

S = 2048
D = 1024
H = 16
DH = 64
DFF = 4096
NKEEP = 64

def _np_pos_encoding():
    pos = np.arange(S)[:, None].astype(np.float32)
    i = np.arange(D)[None, :].astype(np.float32)
    angle = pos / np.power(10000.0, (2.0 * (i // 2)) / D)
    pe = np.zeros((S, D), dtype=np.float32)
    pe[:, 0::2] = np.sin(angle[:, 0::2])
    pe[:, 1::2] = np.cos(angle[:, 1::2])
    return pe


_PE = _np_pos_encoding()


def _embed_body(tok_ref, emb_ref, pe_ref, x_ref):
    x_ref[...] = emb_ref[...] + pe_ref[...]


def _embed(tok, embed, pe):
    out = pl.pallas_call(
        _embed_body,
        grid_spec=pltpu.PrefetchScalarGridSpec(
            num_scalar_prefetch=1,
            grid=(S,),
            in_specs=[
                pl.BlockSpec((1, 1, D), lambda i, tok: (tok[i], 0, 0)),
                pl.BlockSpec((1, 1, D), lambda i, tok: (i, 0, 0)),
            ],
            out_specs=pl.BlockSpec((1, 1, D), lambda i, tok: (i, 0, 0)),
        ),
        out_shape=jax.ShapeDtypeStruct((S, 1, D), jnp.float32),
    )(tok, embed.reshape(embed.shape[0], 1, D), pe.reshape(S, 1, D))
    return out.reshape(S, D)


_BSQKV = 512


def _qkv_body(x_ref, wq_ref, wk_ref, wv_ref, bq_ref, bk_ref, bv_ref,
              q_ref, k_ref, v_ref):
    x = x_ref[...]
    q_ref[...] = jnp.dot(x, wq_ref[...], preferred_element_type=jnp.float32) + bq_ref[...]
    k_ref[...] = jnp.dot(x, wk_ref[...], preferred_element_type=jnp.float32) + bk_ref[...]
    v_ref[...] = jnp.dot(x, wv_ref[...], preferred_element_type=jnp.float32) + bv_ref[...]


def _qkv(x, wq, wk, wv, bq, bk, bv):
    n = S // _BSQKV
    hd = H * DH
    wspec = pl.BlockSpec((D, hd), lambda i: (0, 0))
    bspec = pl.BlockSpec((1, hd), lambda i: (0, 0))
    ospec = pl.BlockSpec((_BSQKV, hd), lambda i: (i, 0))
    out = jax.ShapeDtypeStruct((S, hd), jnp.float32)
    return pl.pallas_call(
        _qkv_body,
        grid=(n,),
        in_specs=[pl.BlockSpec((_BSQKV, D), lambda i: (i, 0)),
                  wspec, wspec, wspec, bspec, bspec, bspec],
        out_specs=[ospec, ospec, ospec],
        out_shape=[out, out, out],
    )(x, wq, wk, wv, bq, bk, bv)


_BQ = 256
_LOW_BIT = 8


def _head_attn(q, k, v):
    s = jax.lax.dot_general(q, k, (((1,), (1,)), ((), ())),
                            preferred_element_type=jnp.float32)
    s = s * jnp.float32(1.0 / 8.0)

    bits = jax.lax.bitcast_convert_type(s, jnp.int32)
    key = jnp.where(bits < 0, bits ^ jnp.int32(0x7FFFFFFF), bits)

    cnt_pos = jnp.sum((key >= 0).astype(jnp.int32), axis=1, keepdims=True)
    prefix = jnp.where(cnt_pos >= NKEEP, jnp.int32(0), jnp.int32(-2147483648))
    for b in range(30, _LOW_BIT - 1, -1):
        cand = prefix | jnp.int32(1 << b)
        cnt = jnp.sum((key >= cand).astype(jnp.int32), axis=1, keepdims=True)
        prefix = jnp.where(cnt >= NKEEP, cand, prefix)

    sm = jnp.where(key >= prefix, s, jnp.float32(-1e9))
    m = jnp.max(sm, axis=1, keepdims=True)
    e = jnp.exp(sm - m)
    p = e / jnp.sum(e, axis=1, keepdims=True)
    return jnp.dot(p, v, preferred_element_type=jnp.float32)


def _attn_body(q_ref, k_ref, v_ref, o_ref):
    for h in range(H):
        sl = slice(h * DH, (h + 1) * DH)
        o_ref[:, sl] = _head_attn(q_ref[:, sl], k_ref[:, sl], v_ref[:, sl])


def _attention(q2d, k2d, v2d):
    nq = S // _BQ
    return pl.pallas_call(
        _attn_body,
        grid=(nq,),
        in_specs=[
            pl.BlockSpec((_BQ, H * DH), lambda iq: (iq, 0)),
            pl.BlockSpec((S, H * DH), lambda iq: (0, 0)),
            pl.BlockSpec((S, H * DH), lambda iq: (0, 0)),
        ],
        out_specs=pl.BlockSpec((_BQ, H * DH), lambda iq: (iq, 0)),
        out_shape=jax.ShapeDtypeStruct((S, H * DH), jnp.float32),
    )(q2d, k2d, v2d)


_BSP = 512


def _post_body(x_ref, o_ref, wo_ref, bo_ref, g_ref, beta_ref, y_ref):
    t = x_ref[...] + jnp.dot(o_ref[...], wo_ref[...],
                             preferred_element_type=jnp.float32) + bo_ref[...]
    mu = jnp.mean(t, axis=1, keepdims=True)
    var = jnp.mean(jnp.square(t - mu), axis=1, keepdims=True)
    y_ref[...] = (t - mu) / jnp.sqrt(var + 1e-5) * g_ref[...] + beta_ref[...]


def _post(x, o, wo, bo, g, beta):
    n = S // _BSP
    vspec = pl.BlockSpec((1, D), lambda i: (0, 0))
    return pl.pallas_call(
        _post_body,
        grid=(n,),
        in_specs=[pl.BlockSpec((_BSP, D), lambda i: (i, 0)),
                  pl.BlockSpec((_BSP, H * DH), lambda i: (i, 0)),
                  pl.BlockSpec((H * DH, D), lambda i: (0, 0)),
                  vspec, vspec, vspec],
        out_specs=pl.BlockSpec((_BSP, D), lambda i: (i, 0)),
        out_shape=jax.ShapeDtypeStruct((S, D), jnp.float32),
    )(x, o, wo, bo, g, beta)


_BSF = 256


def _ffn_body(y_ref, w1_ref, c1_ref, w2_ref, c2_ref, g_ref, beta_ref, z_ref):
    y = y_ref[...]
    h = jnp.maximum(
        jnp.dot(y, w1_ref[...], preferred_element_type=jnp.float32) + c1_ref[...],
        jnp.float32(0.0))
    t = y + jnp.dot(h, w2_ref[...], preferred_element_type=jnp.float32) + c2_ref[...]
    mu = jnp.mean(t, axis=1, keepdims=True)
    var = jnp.mean(jnp.square(t - mu), axis=1, keepdims=True)
    z_ref[...] = (t - mu) / jnp.sqrt(var + 1e-5) * g_ref[...] + beta_ref[...]


def _ffn(y, w1, c1, w2, c2, g, beta):
    n = S // _BSF
    return pl.pallas_call(
        _ffn_body,
        grid=(n,),
        in_specs=[pl.BlockSpec((_BSF, D), lambda i: (i, 0)),
                  pl.BlockSpec((D, DFF), lambda i: (0, 0)),
                  pl.BlockSpec((1, DFF), lambda i: (0, 0)),
                  pl.BlockSpec((DFF, D), lambda i: (0, 0)),
                  pl.BlockSpec((1, D), lambda i: (0, 0)),
                  pl.BlockSpec((1, D), lambda i: (0, 0)),
                  pl.BlockSpec((1, D), lambda i: (0, 0))],
        out_specs=pl.BlockSpec((_BSF, D), lambda i: (i, 0)),
        out_shape=jax.ShapeDtypeStruct((S, D), jnp.float32),
    )(y, w1, c1, w2, c2, g, beta)


@jax.jit
def _forward_impl(tokens, embed, Wq, bq, Wk, bk, Wv, bv, Wo, bo, g1, beta1,
                  W1, c1, W2, c2, g2, beta2):
    tok = tokens.reshape(S).astype(jnp.int32)
    pe = jnp.asarray(_PE)
    x = _embed(tok, embed, pe)
    L = Wq.shape[0]
    for l in range(L):
        q2d, k2d, v2d = _qkv(x, Wq[l], Wk[l], Wv[l],
                             bq[l][None], bk[l][None], bv[l][None])
        o = _attention(q2d, k2d, v2d)
        y = _post(x, o, Wo[l], bo[l][None], g1[l][None], beta1[l][None])
        x = _ffn(y, W1[l], c1[l][None], W2[l], c2[l][None],
                 g2[l][None], beta2[l][None])
    return x[None]


def kernel(tokens, embed, Wq, bq, Wk, bk, Wv, bv, Wo, bo, g1, beta1,
           W1, c1, W2, c2, g2, beta2):
    return _forward_impl(tokens, embed, Wq, bq, Wk, bk, Wv, bv, Wo, bo,
                         g1, beta1, W1, c1, W2, c2, g2, beta2)


# speedup vs baseline: 11.9109x; 1.5931x over previous
import functools

import numpy as np
import jax
import jax.numpy as jnp
from jax import lax
from jax.experimental import pallas as pl
from jax.experimental.pallas import tpu as pltpu
from jax.experimental.pallas import tpu_sc as plsc

S = 2048
D = 1024
H = 16
DH = 64
DFF = 4096
NKEEP = 64

def _np_pos_encoding():
    pos = np.arange(S)[:, None].astype(np.float32)
    i = np.arange(D)[None, :].astype(np.float32)
    angle = pos / np.power(10000.0, (2.0 * (i // 2)) / D)
    pe = np.zeros((S, D), dtype=np.float32)
    pe[:, 0::2] = np.sin(angle[:, 0::2])
    pe[:, 1::2] = np.cos(angle[:, 1::2])
    return pe


_PE = _np_pos_encoding()


_NC = 2
_NS = 16
_NW = _NC * _NS
_BPW = S // _NW


def _sc_gather_body(table_hbm, idx_hbm, out_hbm, idx_v, rows_v, sem):
    wid = lax.axis_index("s") * _NC + lax.axis_index("c")
    base = wid * _BPW
    pltpu.sync_copy(idx_hbm.at[pl.ds(base, _BPW)], idx_v)
    pltpu.async_copy(table_hbm.at[idx_v], rows_v, sem).wait()
    pltpu.sync_copy(rows_v, out_hbm.at[pl.ds(base, _BPW)])


def _embed_gather(table, idx):
    mesh = plsc.VectorSubcoreMesh(core_axis_name="c", subcore_axis_name="s")
    run = functools.partial(
        pl.kernel,
        out_type=jax.ShapeDtypeStruct((S, D), jnp.float32),
        mesh=mesh,
        scratch_types=[
            pltpu.VMEM((_BPW,), jnp.int32),
            pltpu.VMEM((_BPW, D), jnp.float32),
            pltpu.SemaphoreType.DMA,
        ],
    )(_sc_gather_body)
    return run(table, idx)


_BSQKV = 512


def _qkv_body(x_ref, wq_ref, wk_ref, wv_ref, bq_ref, bk_ref, bv_ref,
              q_ref, k_ref, v_ref):
    x = x_ref[...]
    q_ref[...] = jnp.dot(x, wq_ref[...], preferred_element_type=jnp.float32) + bq_ref[...]
    k_ref[...] = jnp.dot(x, wk_ref[...], preferred_element_type=jnp.float32) + bk_ref[...]
    v_ref[...] = jnp.dot(x, wv_ref[...], preferred_element_type=jnp.float32) + bv_ref[...]


def _qkv(x, wq, wk, wv, bq, bk, bv):
    n = S // _BSQKV
    hd = H * DH
    wspec = pl.BlockSpec((D, hd), lambda i: (0, 0))
    bspec = pl.BlockSpec((1, hd), lambda i: (0, 0))
    ospec = pl.BlockSpec((_BSQKV, hd), lambda i: (i, 0))
    out = jax.ShapeDtypeStruct((S, hd), jnp.float32)
    return pl.pallas_call(
        _qkv_body,
        grid=(n,),
        in_specs=[pl.BlockSpec((_BSQKV, D), lambda i: (i, 0)),
                  wspec, wspec, wspec, bspec, bspec, bspec],
        out_specs=[ospec, ospec, ospec],
        out_shape=[out, out, out],
    )(x, wq, wk, wv, bq, bk, bv)


def _qkv_embed_body(emb_ref, pe_ref, wq_ref, wk_ref, wv_ref,
                    bq_ref, bk_ref, bv_ref, x_ref, q_ref, k_ref, v_ref):
    x = emb_ref[...] + pe_ref[...]
    x_ref[...] = x
    q_ref[...] = jnp.dot(x, wq_ref[...], preferred_element_type=jnp.float32) + bq_ref[...]
    k_ref[...] = jnp.dot(x, wk_ref[...], preferred_element_type=jnp.float32) + bk_ref[...]
    v_ref[...] = jnp.dot(x, wv_ref[...], preferred_element_type=jnp.float32) + bv_ref[...]


def _qkv_embed(emb, pe, wq, wk, wv, bq, bk, bv):
    n = S // _BSQKV
    hd = H * DH
    wspec = pl.BlockSpec((D, hd), lambda i: (0, 0))
    bspec = pl.BlockSpec((1, hd), lambda i: (0, 0))
    ospec = pl.BlockSpec((_BSQKV, hd), lambda i: (i, 0))
    out = jax.ShapeDtypeStruct((S, hd), jnp.float32)
    xspec = pl.BlockSpec((_BSQKV, D), lambda i: (i, 0))
    return pl.pallas_call(
        _qkv_embed_body,
        grid=(n,),
        in_specs=[xspec, xspec, wspec, wspec, wspec, bspec, bspec, bspec],
        out_specs=[xspec, ospec, ospec, ospec],
        out_shape=[jax.ShapeDtypeStruct((S, D), jnp.float32), out, out, out],
    )(emb, pe, wq, wk, wv, bq, bk, bv)


_BQ = 256
_LOW_BIT = 12


def _head_attn(q, k, v):
    s = jax.lax.dot_general(q, k, (((1,), (1,)), ((), ())),
                            preferred_element_type=jnp.float32)
    s = s * jnp.float32(1.0 / 8.0)

    bits = jax.lax.bitcast_convert_type(s, jnp.int32)
    key = jnp.where(bits < 0, bits ^ jnp.int32(0x7FFFFFFF), bits)

    cnt_pos = jnp.sum((key >= 0).astype(jnp.int32), axis=1, keepdims=True)
    prefix = jnp.where(cnt_pos >= NKEEP, jnp.int32(0), jnp.int32(-2147483648))
    for b in range(30, _LOW_BIT - 1, -1):
        cand = prefix | jnp.int32(1 << b)
        cnt = jnp.sum((key >= cand).astype(jnp.int32), axis=1, keepdims=True)
        prefix = jnp.where(cnt >= NKEEP, cand, prefix)

    sm = jnp.where(key >= prefix, s, jnp.float32(-1e9))
    m = jnp.max(sm, axis=1, keepdims=True)
    e = jnp.exp(sm - m)
    p = e / jnp.sum(e, axis=1, keepdims=True)
    return jnp.dot(p, v, preferred_element_type=jnp.float32)


def _attn_body(q_ref, k_ref, v_ref, o_ref):
    for h in range(H):
        sl = slice(h * DH, (h + 1) * DH)
        o_ref[:, sl] = _head_attn(q_ref[:, sl], k_ref[:, sl], v_ref[:, sl])


def _attention(q2d, k2d, v2d):
    nq = S // _BQ
    return pl.pallas_call(
        _attn_body,
        grid=(nq,),
        in_specs=[
            pl.BlockSpec((_BQ, H * DH), lambda iq: (iq, 0)),
            pl.BlockSpec((S, H * DH), lambda iq: (0, 0)),
            pl.BlockSpec((S, H * DH), lambda iq: (0, 0)),
        ],
        out_specs=pl.BlockSpec((_BQ, H * DH), lambda iq: (iq, 0)),
        out_shape=jax.ShapeDtypeStruct((S, H * DH), jnp.float32),
    )(q2d, k2d, v2d)


_BSP = 512


def _post_body(x_ref, o_ref, wo_ref, bo_ref, g_ref, beta_ref, y_ref):
    t = x_ref[...] + jnp.dot(o_ref[...], wo_ref[...],
                             preferred_element_type=jnp.float32) + bo_ref[...]
    mu = jnp.mean(t, axis=1, keepdims=True)
    var = jnp.mean(jnp.square(t - mu), axis=1, keepdims=True)
    y_ref[...] = (t - mu) / jnp.sqrt(var + 1e-5) * g_ref[...] + beta_ref[...]


def _post(x, o, wo, bo, g, beta):
    n = S // _BSP
    vspec = pl.BlockSpec((1, D), lambda i: (0, 0))
    return pl.pallas_call(
        _post_body,
        grid=(n,),
        in_specs=[pl.BlockSpec((_BSP, D), lambda i: (i, 0)),
                  pl.BlockSpec((_BSP, H * DH), lambda i: (i, 0)),
                  pl.BlockSpec((H * DH, D), lambda i: (0, 0)),
                  vspec, vspec, vspec],
        out_specs=pl.BlockSpec((_BSP, D), lambda i: (i, 0)),
        out_shape=jax.ShapeDtypeStruct((S, D), jnp.float32),
    )(x, o, wo, bo, g, beta)


_BSF = 256


def _ffn_body(y_ref, w1_ref, c1_ref, w2_ref, c2_ref, g_ref, beta_ref, z_ref):
    y = y_ref[...]
    h = jnp.maximum(
        jnp.dot(y, w1_ref[...], preferred_element_type=jnp.float32) + c1_ref[...],
        jnp.float32(0.0))
    t = y + jnp.dot(h, w2_ref[...], preferred_element_type=jnp.float32) + c2_ref[...]
    mu = jnp.mean(t, axis=1, keepdims=True)
    var = jnp.mean(jnp.square(t - mu), axis=1, keepdims=True)
    z_ref[...] = (t - mu) / jnp.sqrt(var + 1e-5) * g_ref[...] + beta_ref[...]


def _ffn(y, w1, c1, w2, c2, g, beta):
    n = S // _BSF
    return pl.pallas_call(
        _ffn_body,
        grid=(n,),
        in_specs=[pl.BlockSpec((_BSF, D), lambda i: (i, 0)),
                  pl.BlockSpec((D, DFF), lambda i: (0, 0)),
                  pl.BlockSpec((1, DFF), lambda i: (0, 0)),
                  pl.BlockSpec((DFF, D), lambda i: (0, 0)),
                  pl.BlockSpec((1, D), lambda i: (0, 0)),
                  pl.BlockSpec((1, D), lambda i: (0, 0)),
                  pl.BlockSpec((1, D), lambda i: (0, 0))],
        out_specs=pl.BlockSpec((_BSF, D), lambda i: (i, 0)),
        out_shape=jax.ShapeDtypeStruct((S, D), jnp.float32),
    )(y, w1, c1, w2, c2, g, beta)


@jax.jit
def _forward_impl(tokens, embed, Wq, bq, Wk, bk, Wv, bv, Wo, bo, g1, beta1,
                  W1, c1, W2, c2, g2, beta2):
    tok = tokens.reshape(S).astype(jnp.int32)
    pe = jnp.asarray(_PE)
    emb = _embed_gather(embed, tok)
    L = Wq.shape[0]
    x = None
    for l in range(L):
        if l == 0:
            x, q2d, k2d, v2d = _qkv_embed(emb, pe, Wq[l], Wk[l], Wv[l],
                                          bq[l][None], bk[l][None], bv[l][None])
        else:
            q2d, k2d, v2d = _qkv(x, Wq[l], Wk[l], Wv[l],
                                 bq[l][None], bk[l][None], bv[l][None])
        o = _attention(q2d, k2d, v2d)
        y = _post(x, o, Wo[l], bo[l][None], g1[l][None], beta1[l][None])
        x = _ffn(y, W1[l], c1[l][None], W2[l], c2[l][None],
                 g2[l][None], beta2[l][None])
    return x[None]


def kernel(tokens, embed, Wq, bq, Wk, bk, Wv, bv, Wo, bo, g1, beta1,
           W1, c1, W2, c2, g2, beta2):
    return _forward_impl(tokens, embed, Wq, bq, Wk, bk, Wv, bv, Wo, bo,
                         g1, beta1, W1, c1, W2, c2, g2, beta2)
